# baseline (speedup 1.0000x reference)
import functools

import jax
import jax.numpy as jnp
from jax import lax
from jax.experimental import pallas as pl
from jax.experimental.pallas import tpu as pltpu
from jax.experimental.pallas import tpu_sc as plsc


def _sc_gather(table, idx):
    B = idx.shape[0]
    D = table.shape[1]
    info = plsc.get_sparse_core_info()
    NC, NS = info.num_cores, info.num_subcores
    NW = NC * NS
    assert B % (8 * NW) == 0 and D % 16 == 0
    b_per_w = B // NW
    mesh = plsc.VectorSubcoreMesh(core_axis_name="c", subcore_axis_name="s")

    @functools.partial(
        pl.kernel,
        out_type=jax.ShapeDtypeStruct((B, D), jnp.float32),
        mesh=mesh,
        scratch_types=[
            pltpu.VMEM((b_per_w,), jnp.int32),
            pltpu.VMEM((b_per_w, D), jnp.float32),
            pltpu.SemaphoreType.DMA,
        ],
        compiler_params=pltpu.CompilerParams(use_tc_tiling_on_sc=False),
    )
    def gather_kernel(table_hbm, idx_hbm, out_hbm, idx_v, rows_v, sem):
        wid = lax.axis_index("s") * NC + lax.axis_index("c")
        base = wid * b_per_w
        pltpu.sync_copy(idx_hbm.at[pl.ds(base, b_per_w)], idx_v)
        pltpu.async_copy(table_hbm.at[idx_v], rows_v, sem).wait()
        pltpu.sync_copy(rows_v, out_hbm.at[pl.ds(base, b_per_w)])

    return gather_kernel(table, idx)


def _dense_kernel(se_ref, dmf_ref, acf_ref, enf_ref,
                  dm_w1, dm_b1, dm_w2, dm_b2, dm_w3, dm_b3,
                  ac_w1, ac_b1, ac_w2, ac_b2,
                  en_w1, en_b1, en_w2, en_b2, en_w3, en_b3,
                  fw1_se, fw1_dm, fw1_ac, fw1_en, fn_b1,
                  fn_w2, fn_b2, fn_w3, fn_b3, fn_w4, fn_b4,
                  out_ref):
    f32 = jnp.float32
    dot = functools.partial(jnp.dot, preferred_element_type=f32)
    relu = lambda x: jnp.maximum(x, 0.0)

    dm = relu(dot(dmf_ref[...], dm_w1[...]) + dm_b1[...])
    dm = relu(dot(dm, dm_w2[...]) + dm_b2[...])
    dm = dot(dm, dm_w3[...]) + dm_b3[...]

    ac = relu(dot(acf_ref[...], ac_w1[...]) + ac_b1[...])
    ac = dot(ac, ac_w2[...]) + ac_b2[...]

    en = relu(dot(enf_ref[...], en_w1[...]) + en_b1[...])
    en = relu(dot(en, en_w2[...]) + en_b2[...])
    en = dot(en, en_w3[...]) + en_b3[...]

    h = (dot(se_ref[...], fw1_se[...]) + dot(dm, fw1_dm[...])
         + dot(ac, fw1_ac[...]) + dot(en, fw1_en[...]) + fn_b1[...])
    h = relu(h)
    h = relu(dot(h, fn_w2[...]) + fn_b2[...])
    h = relu(dot(h, fn_w3[...]) + fn_b3[...])
    h = dot(h, fn_w4[...]) + fn_b4[...]
    norm = lax.rsqrt(jnp.maximum(jnp.sum(h * h, axis=1, keepdims=True), 1e-12))
    out_ref[...] = h * norm


def _dense_stack(se, dmf, acf, enf, weights, blk):
    B = se.shape[0]
    D = se.shape[1]
    grid = B // blk

    def rows(i):
        return (i, 0)

    def whole(i):
        return (0, 0)

    row_spec = lambda w: pl.BlockSpec((blk, w), rows)
    w_specs = [pl.BlockSpec(w.shape, whole) for w in weights]

    return pl.pallas_call(
        _dense_kernel,
        grid=(grid,),
        in_specs=[row_spec(D), row_spec(3), row_spec(3), row_spec(4)] + w_specs,
        out_specs=pl.BlockSpec((blk, D), rows),
        out_shape=jax.ShapeDtypeStruct((B, D), jnp.float32),
    )(se, dmf, acf, enf, *weights)


def kernel(student_id, demographic_features, academic_scores, engagement_features, emb_table,
           dm_w1, dm_b1, dm_w2, dm_b2, dm_w3, dm_b3,
           ac_w1, ac_b1, ac_w2, ac_b2,
           en_w1, en_b1, en_w2, en_b2, en_w3, en_b3,
           fn_w1, fn_b1, fn_w2, fn_b2, fn_w3, fn_b3, fn_w4, fn_b4):
    D = emb_table.shape[1]
    idx = student_id.astype(jnp.int32)
    se = _sc_gather(emb_table, idx)

    r2 = lambda b: b.reshape(1, -1)
    weights = [
        dm_w1, r2(dm_b1), dm_w2, r2(dm_b2), dm_w3, r2(dm_b3),
        ac_w1, r2(ac_b1), ac_w2, r2(ac_b2),
        en_w1, r2(en_b1), en_w2, r2(en_b2), en_w3, r2(en_b3),
        fn_w1[0:D], fn_w1[D:2 * D], fn_w1[2 * D:3 * D], fn_w1[3 * D:4 * D],
        r2(fn_b1), fn_w2, r2(fn_b2), fn_w3, r2(fn_b3), fn_w4, r2(fn_b4),
    ]
    return _dense_stack(se, demographic_features, academic_scores,
                        engagement_features, weights, blk=2048)


# speedup vs baseline: 1.2895x; 1.2895x over previous
import functools

import jax
import jax.numpy as jnp
from jax import lax
from jax.experimental import pallas as pl
from jax.experimental.pallas import tpu as pltpu
from jax.experimental.pallas import tpu_sc as plsc


def _repack_kernel(lo_ref, hi_ref, out_ref):
    out_ref[...] = jnp.concatenate([lo_ref[...].T, hi_ref[...].T], axis=1)


def _repack(table_t, cols_per_blk, nblk):
    D = table_t.shape[0]
    return pl.pallas_call(
        _repack_kernel,
        grid=(nblk,),
        in_specs=[
            pl.BlockSpec((D, cols_per_blk), lambda i: (0, i)),
            pl.BlockSpec((D, cols_per_blk),
                         lambda i: (0, jnp.minimum(i + nblk, 2 * nblk - 2))),
        ],
        out_specs=pl.BlockSpec((cols_per_blk, 2 * D), lambda i: (i, 0)),
        out_shape=jax.ShapeDtypeStruct(
            (nblk * cols_per_blk, 2 * D), jnp.float32),
    )(table_t, table_t)


def _sc_gather(table, idx):
    B = idx.shape[0]
    W = table.shape[1]
    info = plsc.get_sparse_core_info()
    NC, NS = info.num_cores, info.num_subcores
    NW = NC * NS
    assert B % (8 * NW) == 0 and W % 128 == 0
    b_per_w = B // NW
    mesh = plsc.VectorSubcoreMesh(core_axis_name="c", subcore_axis_name="s")

    @functools.partial(
        pl.kernel,
        out_type=jax.ShapeDtypeStruct((B, W), jnp.float32),
        mesh=mesh,
        scratch_types=[
            pltpu.VMEM((b_per_w,), jnp.int32),
            pltpu.VMEM((b_per_w, W), jnp.float32),
            pltpu.SemaphoreType.DMA,
        ],
    )
    def gather_kernel(table_hbm, idx_hbm, out_hbm, idx_v, rows_v, sem):
        wid = lax.axis_index("s") * NC + lax.axis_index("c")
        base = wid * b_per_w
        pltpu.sync_copy(idx_hbm.at[pl.ds(base, b_per_w)], idx_v)
        pltpu.async_copy(table_hbm.at[idx_v], rows_v, sem).wait()
        pltpu.sync_copy(rows_v, out_hbm.at[pl.ds(base, b_per_w)])

    return gather_kernel(table, idx)


def _dense_kernel(pairs_ref, half_ref, dmf_ref, acf_ref, enf_ref,
                  dm_w1, dm_b1, dm_w2, dm_b2, dm_w3, dm_b3,
                  ac_w1, ac_b1, ac_w2, ac_b2,
                  en_w1, en_b1, en_w2, en_b2, en_w3, en_b3,
                  fw1_se, fw1_dm, fw1_ac, fw1_en, fn_b1,
                  fn_w2, fn_b2, fn_w3, fn_b3, fn_w4, fn_b4,
                  out_ref):
    f32 = jnp.float32
    dot = functools.partial(jnp.dot, preferred_element_type=f32)
    relu = lambda x: jnp.maximum(x, 0.0)

    D = pairs_ref.shape[1] // 2
    g = pairs_ref[...]
    m = half_ref[...] > 0.5
    se = jnp.where(m, g[:, D:], g[:, :D])

    dm = relu(dot(dmf_ref[...], dm_w1[...]) + dm_b1[...])
    dm = relu(dot(dm, dm_w2[...]) + dm_b2[...])
    dm = dot(dm, dm_w3[...]) + dm_b3[...]

    ac = relu(dot(acf_ref[...], ac_w1[...]) + ac_b1[...])
    ac = dot(ac, ac_w2[...]) + ac_b2[...]

    en = relu(dot(enf_ref[...], en_w1[...]) + en_b1[...])
    en = relu(dot(en, en_w2[...]) + en_b2[...])
    en = dot(en, en_w3[...]) + en_b3[...]

    h = (dot(se, fw1_se[...]) + dot(dm, fw1_dm[...])
         + dot(ac, fw1_ac[...]) + dot(en, fw1_en[...]) + fn_b1[...])
    h = relu(h)
    h = relu(dot(h, fn_w2[...]) + fn_b2[...])
    h = relu(dot(h, fn_w3[...]) + fn_b3[...])
    h = dot(h, fn_w4[...]) + fn_b4[...]
    norm = lax.rsqrt(jnp.maximum(jnp.sum(h * h, axis=1, keepdims=True), 1e-12))
    out_ref[...] = h * norm


def _dense_stack(pairs, half, dmf, acf, enf, weights, blk):
    B = pairs.shape[0]
    D = pairs.shape[1] // 2
    grid = B // blk

    def rows(i):
        return (i, 0)

    def whole(i):
        return (0, 0)

    row_spec = lambda w: pl.BlockSpec((blk, w), rows)
    w_specs = [pl.BlockSpec(w.shape, whole) for w in weights]

    return pl.pallas_call(
        _dense_kernel,
        grid=(grid,),
        in_specs=[row_spec(2 * D), row_spec(1), row_spec(3), row_spec(3),
                  row_spec(4)] + w_specs,
        out_specs=pl.BlockSpec((blk, D), rows),
        out_shape=jax.ShapeDtypeStruct((B, D), jnp.float32),
    )(pairs, half, dmf, acf, enf, *weights)


def kernel(student_id, demographic_features, academic_scores, engagement_features, emb_table,
           dm_w1, dm_b1, dm_w2, dm_b2, dm_w3, dm_b3,
           ac_w1, ac_b1, ac_w2, ac_b2,
           en_w1, en_b1, en_w2, en_b2, en_w3, en_b3,
           fn_w1, fn_b1, fn_w2, fn_b2, fn_w3, fn_b3, fn_w4, fn_b4):
    D = emb_table.shape[1]
    N = emb_table.shape[0]
    idx = student_id.astype(jnp.int32)

    table_t = emb_table.T
    cols_per_blk = 1024
    nblk = -(-N // (2 * cols_per_blk))
    M = nblk * cols_per_blk
    pair_table = _repack(table_t, cols_per_blk, nblk)

    in_hi = (idx >= M).astype(jnp.int32)
    pairs = _sc_gather(pair_table, idx - in_hi * M)
    half = in_hi.astype(jnp.float32).reshape(-1, 1)

    r2 = lambda b: b.reshape(1, -1)
    weights = [
        dm_w1, r2(dm_b1), dm_w2, r2(dm_b2), dm_w3, r2(dm_b3),
        ac_w1, r2(ac_b1), ac_w2, r2(ac_b2),
        en_w1, r2(en_b1), en_w2, r2(en_b2), en_w3, r2(en_b3),
        fn_w1[0:D], fn_w1[D:2 * D], fn_w1[2 * D:3 * D], fn_w1[3 * D:4 * D],
        r2(fn_b1), fn_w2, r2(fn_b2), fn_w3, r2(fn_b3), fn_w4, r2(fn_b4),
    ]
    return _dense_stack(pairs, half, demographic_features, academic_scores,
                        engagement_features, weights, blk=2048)


# speedup vs baseline: 2.0231x; 1.5689x over previous
import functools

import jax
import jax.numpy as jnp
from jax import lax
from jax.experimental import pallas as pl
from jax.experimental.pallas import tpu as pltpu
from jax.experimental.pallas import tpu_sc as plsc


def _repack_kernel(lo_ref, hi_ref, out_ref):
    out_ref[...] = jnp.concatenate([lo_ref[...].T, hi_ref[...].T], axis=1)


def _repack(table_t, cols_per_blk, nblk):
    D = table_t.shape[0]
    return pl.pallas_call(
        _repack_kernel,
        grid=(nblk,),
        in_specs=[
            pl.BlockSpec((D, cols_per_blk), lambda i: (0, i)),
            pl.BlockSpec((D, cols_per_blk),
                         lambda i: (0, jnp.minimum(i + nblk, 2 * nblk - 2))),
        ],
        out_specs=pl.BlockSpec((cols_per_blk, 2 * D), lambda i: (i, 0)),
        out_shape=jax.ShapeDtypeStruct(
            (nblk * cols_per_blk, 2 * D), jnp.float32),
    )(table_t, table_t)


def _sc_gather(table, idx):
    B = idx.shape[0]
    W = table.shape[1]
    info = plsc.get_sparse_core_info()
    NC, NS = info.num_cores, info.num_subcores
    NW = NC * NS
    assert B % (8 * NW) == 0 and W % 128 == 0
    b_per_w = B // NW
    mesh = plsc.VectorSubcoreMesh(core_axis_name="c", subcore_axis_name="s")

    @functools.partial(
        pl.kernel,
        out_type=jax.ShapeDtypeStruct((B, W), jnp.float32),
        mesh=mesh,
        scratch_types=[
            pltpu.VMEM((b_per_w,), jnp.int32),
            pltpu.VMEM((b_per_w, W), jnp.float32),
            pltpu.SemaphoreType.DMA,
        ],
    )
    def gather_kernel(table_hbm, idx_hbm, out_hbm, idx_v, rows_v, sem):
        wid = lax.axis_index("s") * NC + lax.axis_index("c")
        base = wid * b_per_w
        pltpu.sync_copy(idx_hbm.at[pl.ds(base, b_per_w)], idx_v)
        pltpu.async_copy(table_hbm.at[idx_v], rows_v, sem).wait()
        pltpu.sync_copy(rows_v, out_hbm.at[pl.ds(base, b_per_w)])

    return gather_kernel(table, idx)


def _dense_kernel(pairs_ref, half_ref, dmf_ref, acf_ref, enf_ref,
                  dm_w1, dm_b1, dm_w2, dm_b2, dm_w3, dm_b3,
                  ac_w1, ac_b1, ac_w2, ac_b2,
                  en_w1, en_b1, en_w2, en_b2, en_w3, en_b3,
                  fw1_se, fw1_dm, fw1_ac, fw1_en, fn_b1,
                  fn_w2, fn_b2, fn_w3, fn_b3, fn_w4, fn_b4,
                  out_ref):
    f32 = jnp.float32
    dot = functools.partial(jnp.dot, preferred_element_type=f32)
    relu = lambda x: jnp.maximum(x, 0.0)

    D = pairs_ref.shape[1] // 2
    g = pairs_ref[...]
    m = half_ref[...] > 0.5
    se = jnp.where(m, g[:, D:], g[:, :D])

    dm = relu(dot(dmf_ref[...], dm_w1[...]) + dm_b1[...])
    dm = relu(dot(dm, dm_w2[...]) + dm_b2[...])
    dm = dot(dm, dm_w3[...]) + dm_b3[...]

    ac = relu(dot(acf_ref[...], ac_w1[...]) + ac_b1[...])
    ac = dot(ac, ac_w2[...]) + ac_b2[...]

    en = relu(dot(enf_ref[...], en_w1[...]) + en_b1[...])
    en = relu(dot(en, en_w2[...]) + en_b2[...])
    en = dot(en, en_w3[...]) + en_b3[...]

    h = (dot(se, fw1_se[...]) + dot(dm, fw1_dm[...])
         + dot(ac, fw1_ac[...]) + dot(en, fw1_en[...]) + fn_b1[...])
    h = relu(h)
    h = relu(dot(h, fn_w2[...]) + fn_b2[...])
    h = relu(dot(h, fn_w3[...]) + fn_b3[...])
    h = dot(h, fn_w4[...]) + fn_b4[...]
    norm = lax.rsqrt(jnp.maximum(jnp.sum(h * h, axis=1, keepdims=True), 1e-12))
    out_ref[...] = h * norm


def _dense_stack(pairs, half, dmf, acf, enf, weights, blk):
    B = pairs.shape[0]
    D = pairs.shape[1] // 2
    grid = B // blk

    def rows(i):
        return (i, 0)

    def whole(i):
        return (0, 0)

    row_spec = lambda w: pl.BlockSpec((blk, w), rows)
    w_specs = [pl.BlockSpec(w.shape, whole) for w in weights]

    return pl.pallas_call(
        _dense_kernel,
        grid=(grid,),
        in_specs=[row_spec(2 * D), row_spec(1), row_spec(3), row_spec(3),
                  row_spec(4)] + w_specs,
        out_specs=pl.BlockSpec((blk, D), rows),
        out_shape=jax.ShapeDtypeStruct((B, D), jnp.float32),
    )(pairs, half, dmf, acf, enf, *weights)


def kernel(student_id, demographic_features, academic_scores, engagement_features, emb_table,
           dm_w1, dm_b1, dm_w2, dm_b2, dm_w3, dm_b3,
           ac_w1, ac_b1, ac_w2, ac_b2,
           en_w1, en_b1, en_w2, en_b2, en_w3, en_b3,
           fn_w1, fn_b1, fn_w2, fn_b2, fn_w3, fn_b3, fn_w4, fn_b4):
    D = emb_table.shape[1]
    N = emb_table.shape[0]
    idx = student_id.astype(jnp.int32)

    table_t = emb_table.T
    cols_per_blk = 4096
    nblk = -(-N // (2 * cols_per_blk))
    M = nblk * cols_per_blk
    pair_table = _repack(table_t, cols_per_blk, nblk)

    in_hi = (idx >= M).astype(jnp.int32)
    pairs = _sc_gather(pair_table, idx - in_hi * M)
    half = in_hi.astype(jnp.float32).reshape(-1, 1)

    r2 = lambda b: b.reshape(1, -1)
    weights = [
        dm_w1, r2(dm_b1), dm_w2, r2(dm_b2), dm_w3, r2(dm_b3),
        ac_w1, r2(ac_b1), ac_w2, r2(ac_b2),
        en_w1, r2(en_b1), en_w2, r2(en_b2), en_w3, r2(en_b3),
        fn_w1[0:D], fn_w1[D:2 * D], fn_w1[2 * D:3 * D], fn_w1[3 * D:4 * D],
        r2(fn_b1), fn_w2, r2(fn_b2), fn_w3, r2(fn_b3), fn_w4, r2(fn_b4),
    ]
    return _dense_stack(pairs, half, demographic_features, academic_scores,
                        engagement_features, weights, blk=2048)


# speedup vs baseline: 2.8513x; 1.4094x over previous
import functools

import jax
import jax.numpy as jnp
from jax import lax
from jax.experimental import pallas as pl
from jax.experimental.pallas import tpu as pltpu
from jax.experimental.pallas import tpu_sc as plsc


def _repack_kernel(a_ref, b_ref, c_ref, d_ref, out_ref):
    u16 = jnp.uint16
    u32 = jnp.uint32
    bf = jnp.bfloat16

    def pk(lo_ref, hi_ref):
        lo = lax.bitcast_convert_type(
            lo_ref[...].astype(bf), u16).astype(u32)
        hi = lax.bitcast_convert_type(
            hi_ref[...].astype(bf), u16).astype(u32)
        return (lo | (hi << 16)).T

    packed = jnp.concatenate([pk(a_ref, b_ref), pk(c_ref, d_ref)], axis=1)
    out_ref[...] = lax.bitcast_convert_type(packed, jnp.float32)


def _repack(table_t, cols_per_blk, nblk, maxblk):
    D = table_t.shape[0]

    def quarter(q):
        return pl.BlockSpec(
            (D, cols_per_blk),
            lambda i, q=q: (0, jnp.minimum(q * nblk + i, maxblk)))

    return pl.pallas_call(
        _repack_kernel,
        grid=(nblk,),
        in_specs=[quarter(0), quarter(1), quarter(2), quarter(3)],
        out_specs=pl.BlockSpec((cols_per_blk, 2 * D), lambda i: (i, 0)),
        out_shape=jax.ShapeDtypeStruct(
            (nblk * cols_per_blk, 2 * D), jnp.float32),
    )(table_t, table_t, table_t, table_t)


def _sc_gather(table, idx):
    B = idx.shape[0]
    W = table.shape[1]
    info = plsc.get_sparse_core_info()
    NC, NS = info.num_cores, info.num_subcores
    NW = NC * NS
    assert B % (8 * NW) == 0 and W % 128 == 0
    b_per_w = B // NW
    mesh = plsc.VectorSubcoreMesh(core_axis_name="c", subcore_axis_name="s")

    @functools.partial(
        pl.kernel,
        out_type=jax.ShapeDtypeStruct((B, W), jnp.float32),
        mesh=mesh,
        scratch_types=[
            pltpu.VMEM((b_per_w,), jnp.int32),
            pltpu.VMEM((b_per_w, W), jnp.float32),
            pltpu.SemaphoreType.DMA,
        ],
    )
    def gather_kernel(table_hbm, idx_hbm, out_hbm, idx_v, rows_v, sem):
        wid = lax.axis_index("s") * NC + lax.axis_index("c")
        base = wid * b_per_w
        pltpu.sync_copy(idx_hbm.at[pl.ds(base, b_per_w)], idx_v)
        pltpu.async_copy(table_hbm.at[idx_v], rows_v, sem).wait()
        pltpu.sync_copy(rows_v, out_hbm.at[pl.ds(base, b_per_w)])

    return gather_kernel(table, idx)


def _dense_kernel(pairs_ref, mlane_ref, mbit_ref, dmf_ref, acf_ref, enf_ref,
                  dm_w1, dm_b1, dm_w2, dm_b2, dm_w3, dm_b3,
                  ac_w1, ac_b1, ac_w2, ac_b2,
                  en_w1, en_b1, en_w2, en_b2, en_w3, en_b3,
                  fw1_se, fw1_dm, fw1_ac, fw1_en, fn_b1,
                  fn_w2, fn_b2, fn_w3, fn_b3, fn_w4, fn_b4,
                  out_ref):
    f32 = jnp.float32
    dot = functools.partial(jnp.dot, preferred_element_type=f32)
    relu = lambda x: jnp.maximum(x, 0.0)

    D = pairs_ref.shape[1] // 2
    u = lax.bitcast_convert_type(pairs_ref[...], jnp.uint32)
    mlane = mlane_ref[...] > 0.5
    mbit = mbit_ref[...] > 0.5
    lanes = jnp.where(mlane, u[:, D:], u[:, :D])
    bits = jnp.where(mbit, lanes & jnp.uint32(0xFFFF0000), lanes << 16)
    se = lax.bitcast_convert_type(bits, f32)

    dm = relu(dot(dmf_ref[...], dm_w1[...]) + dm_b1[...])
    dm = relu(dot(dm, dm_w2[...]) + dm_b2[...])
    dm = dot(dm, dm_w3[...]) + dm_b3[...]

    ac = relu(dot(acf_ref[...], ac_w1[...]) + ac_b1[...])
    ac = dot(ac, ac_w2[...]) + ac_b2[...]

    en = relu(dot(enf_ref[...], en_w1[...]) + en_b1[...])
    en = relu(dot(en, en_w2[...]) + en_b2[...])
    en = dot(en, en_w3[...]) + en_b3[...]

    h = (dot(se, fw1_se[...]) + dot(dm, fw1_dm[...])
         + dot(ac, fw1_ac[...]) + dot(en, fw1_en[...]) + fn_b1[...])
    h = relu(h)
    h = relu(dot(h, fn_w2[...]) + fn_b2[...])
    h = relu(dot(h, fn_w3[...]) + fn_b3[...])
    h = dot(h, fn_w4[...]) + fn_b4[...]
    norm = lax.rsqrt(jnp.maximum(jnp.sum(h * h, axis=1, keepdims=True), 1e-12))
    out_ref[...] = h * norm


def _dense_stack(pairs, mlane, mbit, dmf, acf, enf, weights, blk):
    B = pairs.shape[0]
    D = pairs.shape[1] // 2
    grid = B // blk

    def rows(i):
        return (i, 0)

    def whole(i):
        return (0, 0)

    row_spec = lambda w: pl.BlockSpec((blk, w), rows)
    w_specs = [pl.BlockSpec(w.shape, whole) for w in weights]

    return pl.pallas_call(
        _dense_kernel,
        grid=(grid,),
        in_specs=[row_spec(2 * D), row_spec(1), row_spec(1), row_spec(3),
                  row_spec(3), row_spec(4)] + w_specs,
        out_specs=pl.BlockSpec((blk, D), rows),
        out_shape=jax.ShapeDtypeStruct((B, D), jnp.float32),
    )(pairs, mlane, mbit, dmf, acf, enf, *weights)


def kernel(student_id, demographic_features, academic_scores, engagement_features, emb_table,
           dm_w1, dm_b1, dm_w2, dm_b2, dm_w3, dm_b3,
           ac_w1, ac_b1, ac_w2, ac_b2,
           en_w1, en_b1, en_w2, en_b2, en_w3, en_b3,
           fn_w1, fn_b1, fn_w2, fn_b2, fn_w3, fn_b3, fn_w4, fn_b4):
    D = emb_table.shape[1]
    N = emb_table.shape[0]
    idx = student_id.astype(jnp.int32)

    table_t = emb_table.T
    cols_per_blk = 4096
    nblk = -(-(N - 1) // (4 * cols_per_blk))
    M = nblk * cols_per_blk
    maxblk = (N - 1) // cols_per_blk
    pack_table = _repack(table_t, cols_per_blk, nblk, maxblk)

    q = idx // M
    pairs = _sc_gather(pack_table, idx - q * M)
    mlane = (q >= 2).astype(jnp.float32).reshape(-1, 1)
    mbit = (q & 1).astype(jnp.float32).reshape(-1, 1)

    r2 = lambda b: b.reshape(1, -1)
    weights = [
        dm_w1, r2(dm_b1), dm_w2, r2(dm_b2), dm_w3, r2(dm_b3),
        ac_w1, r2(ac_b1), ac_w2, r2(ac_b2),
        en_w1, r2(en_b1), en_w2, r2(en_b2), en_w3, r2(en_b3),
        fn_w1[0:D], fn_w1[D:2 * D], fn_w1[2 * D:3 * D], fn_w1[3 * D:4 * D],
        r2(fn_b1), fn_w2, r2(fn_b2), fn_w3, r2(fn_b3), fn_w4, r2(fn_b4),
    ]
    return _dense_stack(pairs, mlane, mbit, demographic_features,
                        academic_scores, engagement_features, weights,
                        blk=2048)


# speedup vs baseline: 3.0760x; 1.0788x over previous
import functools

import jax
import jax.numpy as jnp
from jax import lax
from jax.experimental import pallas as pl
from jax.experimental.pallas import tpu as pltpu
from jax.experimental.pallas import tpu_sc as plsc


def _repack_kernel(a_ref, b_ref, c_ref, d_ref, out_ref):
    u16 = jnp.uint16
    u32 = jnp.uint32
    bf = jnp.bfloat16

    def pk(lo_ref, hi_ref):
        lo = lax.bitcast_convert_type(
            lo_ref[...].astype(bf), u16).astype(u32)
        hi = lax.bitcast_convert_type(
            hi_ref[...].astype(bf), u16).astype(u32)
        return (lo | (hi << 16)).T

    packed = jnp.concatenate([pk(a_ref, b_ref), pk(c_ref, d_ref)], axis=1)
    out_ref[...] = lax.bitcast_convert_type(packed, jnp.float32)


def _repack(table_t, cols_per_blk, nblk, maxblk):
    D = table_t.shape[0]

    def quarter(q):
        return pl.BlockSpec(
            (D, cols_per_blk),
            lambda i, q=q: (0, jnp.minimum(q * nblk + i, maxblk)))

    return pl.pallas_call(
        _repack_kernel,
        grid=(nblk,),
        in_specs=[quarter(0), quarter(1), quarter(2), quarter(3)],
        out_specs=pl.BlockSpec((cols_per_blk, 2 * D), lambda i: (i, 0)),
        out_shape=jax.ShapeDtypeStruct(
            (nblk * cols_per_blk, 2 * D), jnp.float32),
    )(table_t, table_t, table_t, table_t)


def _sc_gather(table, idx):
    B = idx.shape[0]
    W = table.shape[1]
    info = plsc.get_sparse_core_info()
    NC, NS = info.num_cores, info.num_subcores
    NW = NC * NS
    assert B % (8 * NW) == 0 and W % 128 == 0
    b_per_w = B // NW
    mesh = plsc.VectorSubcoreMesh(core_axis_name="c", subcore_axis_name="s")

    @functools.partial(
        pl.kernel,
        out_type=jax.ShapeDtypeStruct((B, W), jnp.float32),
        mesh=mesh,
        scratch_types=[
            pltpu.VMEM((b_per_w,), jnp.int32),
            pltpu.VMEM((b_per_w, W), jnp.float32),
            pltpu.SemaphoreType.DMA,
        ],
    )
    def gather_kernel(table_hbm, idx_hbm, out_hbm, idx_v, rows_v, sem):
        wid = lax.axis_index("s") * NC + lax.axis_index("c")
        base = wid * b_per_w
        pltpu.sync_copy(idx_hbm.at[pl.ds(base, b_per_w)], idx_v)
        pltpu.async_copy(table_hbm.at[idx_v], rows_v, sem).wait()
        pltpu.sync_copy(rows_v, out_hbm.at[pl.ds(base, b_per_w)])

    return gather_kernel(table, idx)


def _dense_kernel(pairs_ref, mlane_ref, mbit_ref, dmf_ref, acf_ref, enf_ref,
                  dm_w1, dm_b1, dm_w2, dm_b2, dm_w3, dm_b3,
                  ac_w1, ac_b1, ac_w2, ac_b2,
                  en_w1, en_b1, en_w2, en_b2, en_w3, en_b3,
                  fw1_se, fw1_dm, fw1_ac, fw1_en, fn_b1,
                  fn_w2, fn_b2, fn_w3, fn_b3, fn_w4, fn_b4,
                  out_ref):
    f32 = jnp.float32
    dot = functools.partial(jnp.dot, preferred_element_type=f32)
    relu = lambda x: jnp.maximum(x, 0.0)

    D = pairs_ref.shape[1] // 2
    u = lax.bitcast_convert_type(pairs_ref[...], jnp.uint32)
    mlane = mlane_ref[...] > 0.5
    mbit = mbit_ref[...] > 0.5
    lanes = jnp.where(mlane, u[:, D:], u[:, :D])
    bits = jnp.where(mbit, lanes & jnp.uint32(0xFFFF0000), lanes << 16)
    se = lax.bitcast_convert_type(bits, f32)

    dm = relu(dot(dmf_ref[...], dm_w1[...]) + dm_b1[...])
    dm = relu(dot(dm, dm_w2[...]) + dm_b2[...])
    dm = dot(dm, dm_w3[...]) + dm_b3[...]

    ac = relu(dot(acf_ref[...], ac_w1[...]) + ac_b1[...])
    ac = dot(ac, ac_w2[...]) + ac_b2[...]

    en = relu(dot(enf_ref[...], en_w1[...]) + en_b1[...])
    en = relu(dot(en, en_w2[...]) + en_b2[...])
    en = dot(en, en_w3[...]) + en_b3[...]

    h = (dot(se, fw1_se[...]) + dot(dm, fw1_dm[...])
         + dot(ac, fw1_ac[...]) + dot(en, fw1_en[...]) + fn_b1[...])
    h = relu(h)
    h = relu(dot(h, fn_w2[...]) + fn_b2[...])
    h = relu(dot(h, fn_w3[...]) + fn_b3[...])
    h = dot(h, fn_w4[...]) + fn_b4[...]
    norm = lax.rsqrt(jnp.maximum(jnp.sum(h * h, axis=1, keepdims=True), 1e-12))
    out_ref[...] = h * norm


def _dense_stack(pairs, mlane, mbit, dmf, acf, enf, weights, blk):
    B = pairs.shape[0]
    D = pairs.shape[1] // 2
    grid = B // blk

    def rows(i):
        return (i, 0)

    def whole(i):
        return (0, 0)

    row_spec = lambda w: pl.BlockSpec((blk, w), rows)
    w_specs = [pl.BlockSpec(w.shape, whole) for w in weights]

    return pl.pallas_call(
        _dense_kernel,
        grid=(grid,),
        in_specs=[row_spec(2 * D), row_spec(1), row_spec(1), row_spec(3),
                  row_spec(3), row_spec(4)] + w_specs,
        out_specs=pl.BlockSpec((blk, D), rows),
        out_shape=jax.ShapeDtypeStruct((B, D), jnp.float32),
    )(pairs, mlane, mbit, dmf, acf, enf, *weights)


def kernel(student_id, demographic_features, academic_scores, engagement_features, emb_table,
           dm_w1, dm_b1, dm_w2, dm_b2, dm_w3, dm_b3,
           ac_w1, ac_b1, ac_w2, ac_b2,
           en_w1, en_b1, en_w2, en_b2, en_w3, en_b3,
           fn_w1, fn_b1, fn_w2, fn_b2, fn_w3, fn_b3, fn_w4, fn_b4):
    D = emb_table.shape[1]
    N = emb_table.shape[0]
    idx = student_id.astype(jnp.int32)

    table_t = emb_table.T
    cols_per_blk = 8192
    nblk = -(-(N - 1) // (4 * cols_per_blk))
    M = nblk * cols_per_blk
    maxblk = (N - 1) // cols_per_blk
    pack_table = _repack(table_t, cols_per_blk, nblk, maxblk)

    q = idx // M
    pairs = _sc_gather(pack_table, idx - q * M)
    mlane = (q >= 2).astype(jnp.float32).reshape(-1, 1)
    mbit = (q & 1).astype(jnp.float32).reshape(-1, 1)

    r2 = lambda b: b.reshape(1, -1)
    weights = [
        dm_w1, r2(dm_b1), dm_w2, r2(dm_b2), dm_w3, r2(dm_b3),
        ac_w1, r2(ac_b1), ac_w2, r2(ac_b2),
        en_w1, r2(en_b1), en_w2, r2(en_b2), en_w3, r2(en_b3),
        fn_w1[0:D], fn_w1[D:2 * D], fn_w1[2 * D:3 * D], fn_w1[3 * D:4 * D],
        r2(fn_b1), fn_w2, r2(fn_b2), fn_w3, r2(fn_b3), fn_w4, r2(fn_b4),
    ]
    return _dense_stack(pairs, mlane, mbit, demographic_features,
                        academic_scores, engagement_features, weights,
                        blk=2048)


# speedup vs baseline: 3.1461x; 1.0228x over previous
import functools

import jax
import jax.numpy as jnp
from jax import lax
from jax.experimental import pallas as pl
from jax.experimental.pallas import tpu as pltpu
from jax.experimental.pallas import tpu_sc as plsc


def _repack_kernel(a_ref, b_ref, c_ref, d_ref, out_ref):
    u16 = jnp.uint16
    u32 = jnp.uint32
    bf = jnp.bfloat16

    def pk(lo_ref, hi_ref):
        lo = lax.bitcast_convert_type(
            lo_ref[...].astype(bf), u16).astype(u32)
        hi = lax.bitcast_convert_type(
            hi_ref[...].astype(bf), u16).astype(u32)
        return (lo | (hi << 16)).T

    packed = jnp.concatenate([pk(a_ref, b_ref), pk(c_ref, d_ref)], axis=1)
    out_ref[...] = lax.bitcast_convert_type(packed, jnp.float32)


def _repack(table_t, cols_per_blk, nblk, maxblk):
    D = table_t.shape[0]

    def quarter(q):
        return pl.BlockSpec(
            (D, cols_per_blk),
            lambda i, q=q: (0, jnp.minimum(q * nblk + i, maxblk)))

    return pl.pallas_call(
        _repack_kernel,
        grid=(nblk,),
        in_specs=[quarter(0), quarter(1), quarter(2), quarter(3)],
        out_specs=pl.BlockSpec((cols_per_blk, 2 * D), lambda i: (i, 0)),
        out_shape=jax.ShapeDtypeStruct(
            (nblk * cols_per_blk, 2 * D), jnp.float32),
    )(table_t, table_t, table_t, table_t)


def _sc_gather(table, idx):
    B = idx.shape[0]
    W = table.shape[1]
    info = plsc.get_sparse_core_info()
    NC, NS = info.num_cores, info.num_subcores
    NW = NC * NS
    assert B % (8 * NW) == 0 and W % 128 == 0
    b_per_w = B // NW
    mesh = plsc.VectorSubcoreMesh(core_axis_name="c", subcore_axis_name="s")

    @functools.partial(
        pl.kernel,
        out_type=jax.ShapeDtypeStruct((B, W), jnp.float32),
        mesh=mesh,
        scratch_types=[
            pltpu.VMEM((b_per_w,), jnp.int32),
            pltpu.VMEM((b_per_w, W), jnp.float32),
            pltpu.SemaphoreType.DMA,
        ],
    )
    def gather_kernel(table_hbm, idx_hbm, out_hbm, idx_v, rows_v, sem):
        wid = lax.axis_index("s") * NC + lax.axis_index("c")
        base = wid * b_per_w
        pltpu.sync_copy(idx_hbm.at[pl.ds(base, b_per_w)], idx_v)
        pltpu.async_copy(table_hbm.at[idx_v], rows_v, sem).wait()
        pltpu.sync_copy(rows_v, out_hbm.at[pl.ds(base, b_per_w)])

    return gather_kernel(table, idx)


def _dense_kernel(pairs_ref, mlane_ref, mbit_ref, dmf_ref, acf_ref, enf_ref,
                  dm_w1, dm_b1, dm_w2, dm_b2, dm_w3, dm_b3,
                  ac_w1, ac_b1, ac_w2, ac_b2,
                  en_w1, en_b1, en_w2, en_b2, en_w3, en_b3,
                  fw1_se, fw1_dm, fw1_ac, fw1_en, fn_b1,
                  fn_w2, fn_b2, fn_w3, fn_b3, fn_w4, fn_b4,
                  out_ref):
    f32 = jnp.float32
    dot = functools.partial(jnp.dot, preferred_element_type=f32)
    relu = lambda x: jnp.maximum(x, 0.0)

    D = pairs_ref.shape[1] // 2
    u = lax.bitcast_convert_type(pairs_ref[...], jnp.uint32)
    mlane = mlane_ref[...] > 0.5
    mbit = mbit_ref[...] > 0.5
    lanes = jnp.where(mlane, u[:, D:], u[:, :D])
    bits = jnp.where(mbit, lanes & jnp.uint32(0xFFFF0000), lanes << 16)
    se = lax.bitcast_convert_type(bits, f32)

    dm = relu(dot(dmf_ref[...], dm_w1[...]) + dm_b1[...])
    dm = relu(dot(dm, dm_w2[...]) + dm_b2[...])
    dm = dot(dm, dm_w3[...]) + dm_b3[...]

    ac = relu(dot(acf_ref[...], ac_w1[...]) + ac_b1[...])
    ac = dot(ac, ac_w2[...]) + ac_b2[...]

    en = relu(dot(enf_ref[...], en_w1[...]) + en_b1[...])
    en = relu(dot(en, en_w2[...]) + en_b2[...])
    en = dot(en, en_w3[...]) + en_b3[...]

    h = (dot(se, fw1_se[...]) + dot(dm, fw1_dm[...])
         + dot(ac, fw1_ac[...]) + dot(en, fw1_en[...]) + fn_b1[...])
    h = relu(h)
    h = relu(dot(h, fn_w2[...]) + fn_b2[...])
    h = relu(dot(h, fn_w3[...]) + fn_b3[...])
    h = dot(h, fn_w4[...]) + fn_b4[...]
    norm = lax.rsqrt(jnp.maximum(jnp.sum(h * h, axis=1, keepdims=True), 1e-12))
    out_ref[...] = h * norm


def _dense_stack(pairs, mlane, mbit, dmf, acf, enf, weights, blk):
    B = pairs.shape[0]
    D = pairs.shape[1] // 2
    grid = B // blk

    def rows(i):
        return (i, 0)

    def whole(i):
        return (0, 0)

    row_spec = lambda w: pl.BlockSpec((blk, w), rows)
    w_specs = [pl.BlockSpec(w.shape, whole) for w in weights]

    return pl.pallas_call(
        _dense_kernel,
        grid=(grid,),
        in_specs=[row_spec(2 * D), row_spec(1), row_spec(1), row_spec(3),
                  row_spec(3), row_spec(4)] + w_specs,
        out_specs=pl.BlockSpec((blk, D), rows),
        out_shape=jax.ShapeDtypeStruct((B, D), jnp.float32),
    )(pairs, mlane, mbit, dmf, acf, enf, *weights)


def kernel(student_id, demographic_features, academic_scores, engagement_features, emb_table,
           dm_w1, dm_b1, dm_w2, dm_b2, dm_w3, dm_b3,
           ac_w1, ac_b1, ac_w2, ac_b2,
           en_w1, en_b1, en_w2, en_b2, en_w3, en_b3,
           fn_w1, fn_b1, fn_w2, fn_b2, fn_w3, fn_b3, fn_w4, fn_b4):
    D = emb_table.shape[1]
    N = emb_table.shape[0]
    idx = student_id.astype(jnp.int32)

    table_t = emb_table.T
    cols_per_blk = 12288
    nblk = -(-(N - 1) // (4 * cols_per_blk))
    M = nblk * cols_per_blk
    maxblk = (N - 1) // cols_per_blk
    pack_table = _repack(table_t, cols_per_blk, nblk, maxblk)

    q = idx // M
    pairs = _sc_gather(pack_table, idx - q * M)
    mlane = (q >= 2).astype(jnp.float32).reshape(-1, 1)
    mbit = (q & 1).astype(jnp.float32).reshape(-1, 1)

    r2 = lambda b: b.reshape(1, -1)
    weights = [
        dm_w1, r2(dm_b1), dm_w2, r2(dm_b2), dm_w3, r2(dm_b3),
        ac_w1, r2(ac_b1), ac_w2, r2(ac_b2),
        en_w1, r2(en_b1), en_w2, r2(en_b2), en_w3, r2(en_b3),
        fn_w1[0:D], fn_w1[D:2 * D], fn_w1[2 * D:3 * D], fn_w1[3 * D:4 * D],
        r2(fn_b1), fn_w2, r2(fn_b2), fn_w3, r2(fn_b3), fn_w4, r2(fn_b4),
    ]
    return _dense_stack(pairs, mlane, mbit, demographic_features,
                        academic_scores, engagement_features, weights,
                        blk=2048)
